# Initial kernel scaffold; baseline (speedup 1.0000x reference)
import functools

import jax
import jax.numpy as jnp
from jax import lax
from jax.experimental import pallas as pl
from jax.experimental.pallas import tpu as pltpu
from jax.experimental.pallas import tpu_sc as plsc

NUM_EDGES = 320000
DIM = 128
NC = 2
NS = 16
NW = NC * NS
EDGES_PER_WORKER = NUM_EDGES // NW
CHUNK = 80
NCHUNKS = EDGES_PER_WORKER // CHUNK
LANES = 16


def _edge_decoder(z_hbm, src_hbm, dst_hbm, out_hbm,
                  src_idx_v, dst_idx_v, src_rows, dst_rows, out_v,
                  sem, osem):
    wid = lax.axis_index("s") * NC + lax.axis_index("c")
    base = wid * EDGES_PER_WORKER

    lane_iota = lax.iota(jnp.int32, LANES)

    def chunk_body(ci, carry):
        cbase = pl.multiple_of(base + ci * CHUNK, 8)
        pltpu.sync_copy(src_hbm.at[pl.ds(cbase, CHUNK)], src_idx_v)
        pltpu.sync_copy(dst_hbm.at[pl.ds(cbase, CHUNK)], dst_idx_v)
        cp_s = pltpu.async_copy(z_hbm.at[src_idx_v], src_rows, sem)
        cp_d = pltpu.async_copy(z_hbm.at[dst_idx_v], dst_rows, sem)
        cp_s.wait()
        cp_d.wait()

        for g in range(CHUNK // LANES):
            e_idx = g * LANES + lane_iota

            def dot_body(d2, acc):
                for dd in range(8):
                    d_vec = jnp.full((LANES,), d2 * 8 + dd, dtype=jnp.int32)
                    s = plsc.load_gather(src_rows, [e_idx, d_vec])
                    t = plsc.load_gather(dst_rows, [e_idx, d_vec])
                    acc = acc + s * t
                return acc

            acc = lax.fori_loop(0, DIM // 8, dot_body,
                                jnp.zeros((LANES,), jnp.float32))
            out_v[pl.ds(g * LANES, LANES)] = 1.0 / (1.0 + jnp.exp(-acc))

        pltpu.sync_copy(out_v, out_hbm.at[pl.ds(cbase, CHUNK)])
        return carry

    lax.fori_loop(0, NCHUNKS, chunk_body, 0)


@jax.jit
def _run(z, src, dst):
    mesh = plsc.VectorSubcoreMesh(core_axis_name="c", subcore_axis_name="s")
    return pl.kernel(
        _edge_decoder,
        out_type=jax.ShapeDtypeStruct((NUM_EDGES,), jnp.float32),
        mesh=mesh,
        scratch_types=[
            pltpu.VMEM((CHUNK,), jnp.int32),
            pltpu.VMEM((CHUNK,), jnp.int32),
            pltpu.VMEM((CHUNK, DIM), jnp.float32),
            pltpu.VMEM((CHUNK, DIM), jnp.float32),
            pltpu.VMEM((CHUNK,), jnp.float32),
            pltpu.SemaphoreType.DMA,
            pltpu.SemaphoreType.DMA,
        ],
    )(z, src, dst)


def kernel(z, edge_index):
    edge_index = edge_index.astype(jnp.int32)
    return _run(z, edge_index[0], edge_index[1])


# speedup vs baseline: 2.4700x; 2.4700x over previous
import functools

import jax
import jax.numpy as jnp
from jax import lax
from jax.experimental import pallas as pl
from jax.experimental.pallas import tpu as pltpu
from jax.experimental.pallas import tpu_sc as plsc

NUM_EDGES = 320000
DIM = 128
NC = 2
NS = 16
NW = NC * NS
EDGES_PER_WORKER = NUM_EDGES // NW
CHUNK = 80
NCHUNKS = EDGES_PER_WORKER // CHUNK
LANES = 16


def _edge_decoder(z_hbm, src_hbm, dst_hbm, out_hbm,
                  src_idx_v, dst_idx_v, src_rows, dst_rows, part_v, out_v,
                  sem, osem):
    wid = lax.axis_index("s") * NC + lax.axis_index("c")
    base = wid * EDGES_PER_WORKER

    lane_iota = lax.iota(jnp.int32, LANES)

    def chunk_body(ci, carry):
        cbase = pl.multiple_of(base + ci * CHUNK, 8)
        pltpu.sync_copy(src_hbm.at[pl.ds(cbase, CHUNK)], src_idx_v)
        pltpu.sync_copy(dst_hbm.at[pl.ds(cbase, CHUNK)], dst_idx_v)
        cp_s = pltpu.async_copy(z_hbm.at[src_idx_v], src_rows, sem)
        cp_d = pltpu.async_copy(z_hbm.at[dst_idx_v], dst_rows, sem)
        cp_s.wait()
        cp_d.wait()

        for g in range(CHUNK // LANES):
            for e in range(LANES):
                row = g * LANES + e
                p = (src_rows[row, pl.ds(0, LANES)]
                     * dst_rows[row, pl.ds(0, LANES)])
                for dd in range(1, DIM // LANES):
                    p = p + (src_rows[row, pl.ds(dd * LANES, LANES)]
                             * dst_rows[row, pl.ds(dd * LANES, LANES)])
                part_v[pl.ds(e * LANES, LANES)] = p

            acc = jnp.zeros((LANES,), jnp.float32)
            for l in range(LANES):
                acc = acc + plsc.load_gather(part_v, [lane_iota * LANES + l])
            out_v[pl.ds(g * LANES, LANES)] = 1.0 / (1.0 + jnp.exp(-acc))

        pltpu.sync_copy(out_v, out_hbm.at[pl.ds(cbase, CHUNK)])
        return carry

    lax.fori_loop(0, NCHUNKS, chunk_body, 0)


@jax.jit
def _run(z, src, dst):
    mesh = plsc.VectorSubcoreMesh(core_axis_name="c", subcore_axis_name="s")
    return pl.kernel(
        _edge_decoder,
        out_type=jax.ShapeDtypeStruct((NUM_EDGES,), jnp.float32),
        mesh=mesh,
        scratch_types=[
            pltpu.VMEM((CHUNK,), jnp.int32),
            pltpu.VMEM((CHUNK,), jnp.int32),
            pltpu.VMEM((CHUNK, DIM), jnp.float32),
            pltpu.VMEM((CHUNK, DIM), jnp.float32),
            pltpu.VMEM((LANES * LANES,), jnp.float32),
            pltpu.VMEM((CHUNK,), jnp.float32),
            pltpu.SemaphoreType.DMA,
            pltpu.SemaphoreType.DMA,
        ],
        compiler_params=pltpu.CompilerParams(needs_layout_passes=False),
    )(z, src, dst)


def kernel(z, edge_index):
    edge_index = edge_index.astype(jnp.int32)
    return _run(z, edge_index[0], edge_index[1])
